# baseline (speedup 1.0000x reference)
import functools

import jax
import jax.numpy as jnp
from jax import lax
from jax.experimental import pallas as pl
from jax.experimental.pallas import tpu as pltpu
from jax.experimental.pallas import tpu_sc as plsc

VOCAB = 100000
D = 128
SEQ = 200
BATCH = 1024
TOKENS = BATCH * SEQ
EPS = 1e-12

_INFO = plsc.get_sparse_core_info()
_NC = _INFO.num_cores
_NS = _INFO.num_subcores
_NW = _NC * _NS
_PER_W = TOKENS // _NW
_CHUNK = 400
_NITER = _PER_W // _CHUNK


def _sc_gather(idx_flat, table):
    mesh = plsc.VectorSubcoreMesh(core_axis_name="c", subcore_axis_name="s")

    @functools.partial(
        pl.kernel,
        mesh=mesh,
        out_type=jax.ShapeDtypeStruct((TOKENS, D), jnp.float32),
        scratch_types=[
            pltpu.VMEM((_CHUNK,), jnp.int32),
            pltpu.VMEM((_CHUNK,), jnp.int32),
            pltpu.VMEM((_CHUNK, D), jnp.float32),
            pltpu.VMEM((_CHUNK, D), jnp.float32),
            pltpu.SemaphoreType.DMA,
            pltpu.SemaphoreType.DMA,
            pltpu.SemaphoreType.DMA,
            pltpu.SemaphoreType.DMA,
        ],
    )
    def k(idx_hbm, table_hbm, out_hbm, idx0, idx1, rows0, rows1,
          gs0, gs1, ws0, ws1):
        wid = lax.axis_index("s") * _NC + lax.axis_index("c")
        base = wid * _PER_W

        idxs = [idx0, idx1]
        bufs = [rows0, rows1]
        gsems = [gs0, gs1]
        wsems = [ws0, ws1]
        g = [None, None]
        w = [None] * _NITER

        pltpu.sync_copy(idx_hbm.at[pl.ds(base, _CHUNK)], idxs[0])
        g[0] = pltpu.async_copy(table_hbm.at[idxs[0]], bufs[0], gsems[0])

        for i in range(_NITER):
            cur = i % 2
            if i + 1 < _NITER:
                nxt = (i + 1) % 2
                if i >= 1:
                    w[i - 1].wait()
                off_n = base + (i + 1) * _CHUNK
                pltpu.sync_copy(idx_hbm.at[pl.ds(off_n, _CHUNK)], idxs[nxt])
                g[nxt] = pltpu.async_copy(
                    table_hbm.at[idxs[nxt]], bufs[nxt], gsems[nxt])
            g[cur].wait()
            off = base + i * _CHUNK
            w[i] = pltpu.async_copy(
                bufs[cur], out_hbm.at[pl.ds(off, _CHUNK)], wsems[cur])

        w[_NITER - 2].wait()
        w[_NITER - 1].wait()

    return k(idx_flat, table)


def _tc_body(rows_ref, tt_ref, pos_ref, type_ref, gamma_ref, beta_ref, out_ref):
    x = rows_ref[...]
    tt = tt_ref[...]
    pos = pos_ref[...]
    t0 = type_ref[0, :]
    t1 = type_ref[1, :]
    te = jnp.where((tt[..., None] == 0), t0[None, None, :], t1[None, None, :])
    x = x + pos[None, :, :] + te
    bb = x.shape[0]
    x2 = x.reshape(bb * SEQ, D)
    ones = jnp.ones((D, D), jnp.float32)
    mean = jnp.dot(x2, ones, preferred_element_type=jnp.float32) * (1.0 / D)
    msq = jnp.dot(x2 * x2, ones, preferred_element_type=jnp.float32) * (1.0 / D)
    var = msq - mean * mean
    y = (x2 - mean) * lax.rsqrt(var + EPS)
    y = y * gamma_ref[...] + beta_ref[...]
    out_ref[...] = y.reshape(bb, SEQ, D)


def _tc_add_ln(rows, token_type_ids, pos_emb, type_emb, gamma, beta):
    BB = 32
    grid = (BATCH // BB,)
    return pl.pallas_call(
        _tc_body,
        grid=grid,
        in_specs=[
            pl.BlockSpec((BB, SEQ, D), lambda i: (i, 0, 0)),
            pl.BlockSpec((BB, SEQ), lambda i: (i, 0)),
            pl.BlockSpec((SEQ, D), lambda i: (0, 0)),
            pl.BlockSpec((2, D), lambda i: (0, 0)),
            pl.BlockSpec((D,), lambda i: (0,)),
            pl.BlockSpec((D,), lambda i: (0,)),
        ],
        out_specs=pl.BlockSpec((BB, SEQ, D), lambda i: (i, 0, 0)),
        out_shape=jax.ShapeDtypeStruct((BATCH, SEQ, D), jnp.float32),
    )(rows, token_type_ids, pos_emb, type_emb, gamma, beta)


def kernel(input_ids, token_type_ids, word_emb, pos_emb, type_emb, gamma, beta):
    idx_flat = input_ids.reshape(TOKENS).astype(jnp.int32)
    rows = _sc_gather(idx_flat, word_emb)
    rows = rows.reshape(BATCH, SEQ, D)
    tt = token_type_ids.astype(jnp.int32)
    pos = pos_emb[:SEQ]
    return _tc_add_ln(rows, tt, pos, type_emb, gamma, beta)


# speedup vs baseline: 1.0566x; 1.0566x over previous
import functools

import jax
import jax.numpy as jnp
from jax import lax
from jax.experimental import pallas as pl
from jax.experimental.pallas import tpu as pltpu
from jax.experimental.pallas import tpu_sc as plsc

VOCAB = 100000
D = 128
SEQ = 200
BATCH = 1024
TOKENS = BATCH * SEQ
EPS = 1e-12

_INFO = plsc.get_sparse_core_info()
_NC = _INFO.num_cores
_NS = _INFO.num_subcores
_NW = _NC * _NS
_PER_W = TOKENS // _NW
_CHUNK = 400
_NITER = _PER_W // _CHUNK


def _sc_gather(idx_flat, table):
    mesh = plsc.VectorSubcoreMesh(core_axis_name="c", subcore_axis_name="s")

    @functools.partial(
        pl.kernel,
        mesh=mesh,
        out_type=jax.ShapeDtypeStruct((TOKENS, D), jnp.float32),
        scratch_types=[
            pltpu.VMEM((_CHUNK,), jnp.int32),
            pltpu.VMEM((_CHUNK,), jnp.int32),
            pltpu.VMEM((_CHUNK, D), jnp.float32),
            pltpu.VMEM((_CHUNK, D), jnp.float32),
            pltpu.SemaphoreType.DMA,
            pltpu.SemaphoreType.DMA,
            pltpu.SemaphoreType.DMA,
            pltpu.SemaphoreType.DMA,
        ],
    )
    def k(idx_hbm, table_hbm, out_hbm, idx0, idx1, rows0, rows1,
          gs0, gs1, ws0, ws1):
        wid = lax.axis_index("s") * _NC + lax.axis_index("c")
        base = wid * _PER_W

        idxs = [idx0, idx1]
        bufs = [rows0, rows1]
        gsems = [gs0, gs1]
        wsems = [ws0, ws1]
        g = [None, None]
        w = [None] * _NITER

        pltpu.sync_copy(idx_hbm.at[pl.ds(base, _CHUNK)], idxs[0])
        g[0] = pltpu.async_copy(table_hbm.at[idxs[0]], bufs[0], gsems[0])

        for i in range(_NITER):
            cur = i % 2
            if i + 1 < _NITER:
                nxt = (i + 1) % 2
                if i >= 1:
                    w[i - 1].wait()
                off_n = base + (i + 1) * _CHUNK
                pltpu.sync_copy(idx_hbm.at[pl.ds(off_n, _CHUNK)], idxs[nxt])
                g[nxt] = pltpu.async_copy(
                    table_hbm.at[idxs[nxt]], bufs[nxt], gsems[nxt])
            g[cur].wait()
            off = base + i * _CHUNK
            w[i] = pltpu.async_copy(
                bufs[cur], out_hbm.at[pl.ds(off, _CHUNK)], wsems[cur])

        w[_NITER - 2].wait()
        w[_NITER - 1].wait()

    return k(idx_flat, table)


def _tc_body(rows_ref, tt_ref, pos_ref, type_ref, gamma_ref, beta_ref, out_ref):
    x = rows_ref[...]
    tt = tt_ref[...]
    pos = pos_ref[...]
    t0 = type_ref[0, :]
    t1 = type_ref[1, :]
    te = jnp.where((tt[..., None] == 0), t0[None, None, :], t1[None, None, :])
    x = x + pos[None, :, :] + te
    bb = x.shape[0]
    x2 = x.reshape(bb * SEQ, D)
    ones = jnp.ones((D, D), jnp.float32)
    mean = jnp.dot(x2, ones, preferred_element_type=jnp.float32) * (1.0 / D)
    msq = jnp.dot(x2 * x2, ones, preferred_element_type=jnp.float32) * (1.0 / D)
    var = msq - mean * mean
    y = (x2 - mean) * lax.rsqrt(var + EPS)
    y = y * gamma_ref[...] + beta_ref[...]
    out_ref[...] = y.reshape(bb, SEQ, D)


def _tc_add_ln(rows, token_type_ids, pos_emb, type_emb, gamma, beta):
    BB = 64
    grid = (BATCH // BB,)
    return pl.pallas_call(
        _tc_body,
        grid=grid,
        in_specs=[
            pl.BlockSpec((BB, SEQ, D), lambda i: (i, 0, 0)),
            pl.BlockSpec((BB, SEQ), lambda i: (i, 0)),
            pl.BlockSpec((SEQ, D), lambda i: (0, 0)),
            pl.BlockSpec((2, D), lambda i: (0, 0)),
            pl.BlockSpec((D,), lambda i: (0,)),
            pl.BlockSpec((D,), lambda i: (0,)),
        ],
        out_specs=pl.BlockSpec((BB, SEQ, D), lambda i: (i, 0, 0)),
        out_shape=jax.ShapeDtypeStruct((BATCH, SEQ, D), jnp.float32),
    )(rows, token_type_ids, pos_emb, type_emb, gamma, beta)


def kernel(input_ids, token_type_ids, word_emb, pos_emb, type_emb, gamma, beta):
    idx_flat = input_ids.reshape(TOKENS).astype(jnp.int32)
    rows = _sc_gather(idx_flat, word_emb)
    rows = rows.reshape(BATCH, SEQ, D)
    tt = token_type_ids.astype(jnp.int32)
    pos = pos_emb[:SEQ]
    return _tc_add_ln(rows, tt, pos, type_emb, gamma, beta)


# speedup vs baseline: 1.0644x; 1.0074x over previous
import functools

import jax
import jax.numpy as jnp
from jax import lax
from jax.experimental import pallas as pl
from jax.experimental.pallas import tpu as pltpu
from jax.experimental.pallas import tpu_sc as plsc

VOCAB = 100000
D = 128
SEQ = 200
BATCH = 1024
TOKENS = BATCH * SEQ
EPS = 1e-12

_INFO = plsc.get_sparse_core_info()
_NC = _INFO.num_cores
_NS = _INFO.num_subcores
_NW = _NC * _NS
_PER_W = TOKENS // _NW
_CHUNK = 400
_NITER = _PER_W // _CHUNK


def _sc_gather(idx_flat, table):
    mesh = plsc.VectorSubcoreMesh(core_axis_name="c", subcore_axis_name="s")

    @functools.partial(
        pl.kernel,
        mesh=mesh,
        out_type=jax.ShapeDtypeStruct((TOKENS, D), jnp.float32),
        scratch_types=[
            pltpu.VMEM((_PER_W,), jnp.int32),
            pltpu.VMEM((_CHUNK, D), jnp.float32),
            pltpu.VMEM((_CHUNK, D), jnp.float32),
            pltpu.SemaphoreType.DMA,
            pltpu.SemaphoreType.DMA,
            pltpu.SemaphoreType.DMA,
            pltpu.SemaphoreType.DMA,
        ],
    )
    def k(idx_hbm, table_hbm, out_hbm, idx_all, rows0, rows1,
          gs0, gs1, ws0, ws1):
        wid = lax.axis_index("s") * _NC + lax.axis_index("c")
        base = wid * _PER_W

        bufs = [rows0, rows1]
        gsems = [gs0, gs1]
        wsems = [ws0, ws1]
        g = [None, None]
        w = [None] * _NITER

        pltpu.sync_copy(idx_hbm.at[pl.ds(base, _PER_W)], idx_all)
        g[0] = pltpu.async_copy(
            table_hbm.at[idx_all.at[pl.ds(0, _CHUNK)]], bufs[0], gsems[0])

        for i in range(_NITER):
            cur = i % 2
            if i + 1 < _NITER:
                nxt = (i + 1) % 2
                if i >= 1:
                    w[i - 1].wait()
                g[nxt] = pltpu.async_copy(
                    table_hbm.at[idx_all.at[pl.ds((i + 1) * _CHUNK, _CHUNK)]],
                    bufs[nxt], gsems[nxt])
            g[cur].wait()
            off = base + i * _CHUNK
            w[i] = pltpu.async_copy(
                bufs[cur], out_hbm.at[pl.ds(off, _CHUNK)], wsems[cur])

        w[_NITER - 2].wait()
        w[_NITER - 1].wait()

    return k(idx_flat, table)


def _tc_body(rows_ref, tt_ref, pos_ref, type_ref, gamma_ref, beta_ref, out_ref):
    x = rows_ref[...]
    tt = tt_ref[...]
    pos = pos_ref[...]
    t0 = type_ref[0, :]
    t1 = type_ref[1, :]
    te = jnp.where((tt[..., None] == 0), t0[None, None, :], t1[None, None, :])
    x = x + pos[None, :, :] + te
    bb = x.shape[0]
    x2 = x.reshape(bb * SEQ, D)
    ones = jnp.ones((D, D), jnp.float32)
    mean = jnp.dot(x2, ones, preferred_element_type=jnp.float32) * (1.0 / D)
    msq = jnp.dot(x2 * x2, ones, preferred_element_type=jnp.float32) * (1.0 / D)
    var = msq - mean * mean
    y = (x2 - mean) * lax.rsqrt(var + EPS)
    y = y * gamma_ref[...] + beta_ref[...]
    out_ref[...] = y.reshape(bb, SEQ, D)


def _tc_add_ln(rows, token_type_ids, pos_emb, type_emb, gamma, beta):
    BB = 64
    grid = (BATCH // BB,)
    return pl.pallas_call(
        _tc_body,
        grid=grid,
        in_specs=[
            pl.BlockSpec((BB, SEQ, D), lambda i: (i, 0, 0)),
            pl.BlockSpec((BB, SEQ), lambda i: (i, 0)),
            pl.BlockSpec((SEQ, D), lambda i: (0, 0)),
            pl.BlockSpec((2, D), lambda i: (0, 0)),
            pl.BlockSpec((D,), lambda i: (0,)),
            pl.BlockSpec((D,), lambda i: (0,)),
        ],
        out_specs=pl.BlockSpec((BB, SEQ, D), lambda i: (i, 0, 0)),
        out_shape=jax.ShapeDtypeStruct((BATCH, SEQ, D), jnp.float32),
    )(rows, token_type_ids, pos_emb, type_emb, gamma, beta)


def kernel(input_ids, token_type_ids, word_emb, pos_emb, type_emb, gamma, beta):
    idx_flat = input_ids.reshape(TOKENS).astype(jnp.int32)
    rows = _sc_gather(idx_flat, word_emb)
    rows = rows.reshape(BATCH, SEQ, D)
    tt = token_type_ids.astype(jnp.int32)
    pos = pos_emb[:SEQ]
    return _tc_add_ln(rows, tt, pos, type_emb, gamma, beta)
